# baseline (speedup 1.0000x reference)
import functools

import jax
import jax.numpy as jnp
from jax import lax
from jax.experimental import pallas as pl
from jax.experimental.pallas import tpu as pltpu
from jax.experimental.pallas import tpu_sc as plsc

BB = 8
SV = 197
SP = 256
NV = BB * SV
NT = 2048
D = 768
NH = 12
DK = 64
F = 1024
EE = 16
CAP = 197
CAPP = 208
NSLOT = EE * CAPP
TRASH = NSLOT
XR = NSLOT + 8
NW = 32
TPW = NT // NW
NCLS = 1000
NCLSP = 1024
NEG = -1e30

_f32 = jnp.float32
_bf16 = jnp.bfloat16


def _dot(a, b):
    return jnp.dot(a.astype(_bf16), b.astype(_bf16),
                   preferred_element_type=_f32)


def _embed_body(p_ref, w_ref, b_ref, o_ref):
    o_ref[...] = _dot(p_ref[...], w_ref[...]) + b_ref[...]


def _embed(patches, W, b):
    return pl.pallas_call(
        _embed_body,
        out_shape=jax.ShapeDtypeStruct((BB * 196, D), _f32),
    )(patches, W, b)


def _ln_body(h_ref, s_ref, b_ref, o_ref):
    hb = h_ref[...]
    m = jnp.mean(hb, -1, keepdims=True)
    v = jnp.mean((hb - m) ** 2, -1, keepdims=True)
    o_ref[...] = (hb - m) / jnp.sqrt(v + 1e-6) * s_ref[...] + b_ref[...]


def _ln(h, s, b):
    return pl.pallas_call(
        _ln_body,
        out_shape=jax.ShapeDtypeStruct((BB, SV, D), _f32),
    )(h, s, b)


def _qkv_body(hn_ref, wq_ref, bq_ref, wk_ref, bk_ref, wv_ref, bv_ref,
              q_ref, k_ref, v_ref):
    hn = hn_ref[...]
    q_ref[...] = _dot(hn, wq_ref[...]) + bq_ref[...]
    k_ref[...] = _dot(hn, wk_ref[...]) + bk_ref[...]
    v_ref[...] = _dot(hn, wv_ref[...]) + bv_ref[...]


def _qkv(hn, Wq, bq, Wk, bk, Wv, bv):
    sh = jax.ShapeDtypeStruct((NT, D), _f32)
    return pl.pallas_call(
        _qkv_body,
        out_shape=[sh, sh, sh],
    )(hn, Wq, bq, Wk, bk, Wv, bv)


def _proj_body(h_ref, o_ref, wo_ref, bo_ref, out_ref):
    out_ref[...] = h_ref[...] + (_dot(o_ref[...], wo_ref[...]) + bo_ref[...])


def _proj(h, o, Wo, bo):
    return pl.pallas_call(
        _proj_body,
        out_shape=jax.ShapeDtypeStruct((NT, D), _f32),
    )(h, o, Wo, bo)


def _gate_body(hn_ref, g_ref, o_ref):
    o_ref[...] = _dot(hn_ref[...], g_ref[...])


def _gate(hn, gate_w):
    return pl.pallas_call(
        _gate_body,
        out_shape=jax.ShapeDtypeStruct((NT, EE), _f32),
    )(hn, gate_w)


def _attn_body(q_ref, k_ref, v_ref, o_ref):
    q = q_ref[0]
    k = k_ref[0]
    vv = v_ref[0]
    kmask = jnp.where(
        lax.broadcasted_iota(jnp.int32, (1, SP), 1) < SV, 0.0, NEG)
    outs = []
    for hd in range(NH):
        sl = slice(hd * DK, (hd + 1) * DK)
        qh, kh, vh = q[:, sl], k[:, sl], vv[:, sl]
        sc = lax.dot_general(qh.astype(_bf16), kh.astype(_bf16),
                             (((1,), (1,)), ((), ())),
                             preferred_element_type=_f32) * 0.125 + kmask
        mx = jnp.max(sc, -1, keepdims=True)
        e = jnp.exp(sc - mx)
        s = (jnp.sum(e[:, :128], -1, keepdims=True)
             + jnp.sum(e[:, 128:], -1, keepdims=True))
        p = e / s
        outs.append(_dot(p, vh))
    o_ref[0] = jnp.concatenate(outs, axis=-1)


def _attncore(q, k, v):
    blk = pl.BlockSpec((1, SP, D), lambda i: (i, 0, 0))
    return pl.pallas_call(
        _attn_body,
        grid=(BB,),
        in_specs=[blk, blk, blk],
        out_specs=blk,
        out_shape=jax.ShapeDtypeStruct((BB, SP, D), _f32),
    )(q.reshape(BB, SP, D), k.reshape(BB, SP, D), v.reshape(BB, SP, D))


def _rgb_body(lg_ref, d1_ref, d2_ref, c1_ref, c2_ref, g1_ref, g2_ref):
    lg = lg_ref[...]
    it = lax.broadcasted_iota(jnp.int32, (NT, EE), 1)
    rowid = lax.broadcasted_iota(jnp.int32, (NT, EE), 0)
    valid = (rowid % SP) < SV
    p = jnp.exp(lg - jnp.max(lg, -1, keepdims=True))
    p = p / jnp.sum(p, -1, keepdims=True)
    mx1 = jnp.max(p, -1, keepdims=True)
    e1 = jnp.min(jnp.where(p == mx1, it, EE), -1)
    sel1 = it == e1[:, None]
    m1 = jnp.where(sel1 & valid, 1.0, 0.0)
    p2m = jnp.where(sel1, 0.0, p)
    mx2 = jnp.max(p2m, -1, keepdims=True)
    e2 = jnp.min(jnp.where(p2m == mx2, it, EE), -1)
    m2 = jnp.where((it == e2[:, None]) & valid, 1.0, 0.0)

    ri = lax.broadcasted_iota(jnp.int32, (256, 256), 0)
    ci = lax.broadcasted_iota(jnp.int32, (256, 256), 1)
    T = jnp.where(ri > ci, 1.0, 0.0)

    def exclusive_cumsum(mm):
        locs = []
        carry = jnp.zeros((1, EE), _f32)
        for bb in range(NT // 256):
            mb = mm[bb * 256:(bb + 1) * 256]
            locs.append(_dot(T, mb) + carry)
            carry = carry + jnp.sum(mb, 0, keepdims=True)
        return jnp.concatenate(locs, 0), carry

    loc1, tot1 = exclusive_cumsum(m1)
    loc2, _ = exclusive_cumsum(m2)
    loc2 = loc2 + tot1
    m1c = jnp.where(loc1 < CAP, m1, 0.0)
    m2c = jnp.where(loc2 < CAP, m2, 0.0)
    p1 = jnp.sum(loc1 * m1c, -1).astype(jnp.int32)
    p2 = jnp.sum(loc2 * m2c, -1).astype(jnp.int32)
    g1 = jnp.sum(p * m1c, -1)
    g2 = jnp.sum(p * m2c, -1)
    den = g1 + g2 + 1e-9
    g1_ref[...] = g1 / den
    g2_ref[...] = g2 / den
    k1 = jnp.sum(m1c, -1) > 0
    k2 = jnp.sum(m2c, -1) > 0
    s1 = e1 * CAPP + p1
    s2 = e2 * CAPP + p2
    d1_ref[...] = jnp.where(k1, s1, TRASH).astype(jnp.int32)
    d2_ref[...] = jnp.where(k2, s2, TRASH).astype(jnp.int32)
    c1_ref[...] = jnp.where(k1, s1, 0).astype(jnp.int32)
    c2_ref[...] = jnp.where(k2, s2, 0).astype(jnp.int32)


def _rgb(lg):
    i32 = jnp.int32
    return pl.pallas_call(
        _rgb_body,
        out_shape=[jax.ShapeDtypeStruct((NT,), i32),
                   jax.ShapeDtypeStruct((NT,), i32),
                   jax.ShapeDtypeStruct((NT,), i32),
                   jax.ShapeDtypeStruct((NT,), i32),
                   jax.ShapeDtypeStruct((NT,), _f32),
                   jax.ShapeDtypeStruct((NT,), _f32)],
    )(lg)


def _ffn_body(x_ref, w1_ref, b1_ref, w2_ref, b2_ref, o_ref):
    xb = x_ref[0]
    h1 = jax.nn.gelu(_dot(xb, w1_ref[0]) + b1_ref[0])
    o_ref[0] = _dot(h1, w2_ref[0]) + b2_ref[0]


def _ffn(X, W1, b1, W2, b2):
    return pl.pallas_call(
        _ffn_body,
        grid=(EE,),
        in_specs=[pl.BlockSpec((1, CAPP, D), lambda i: (i, 0, 0)),
                  pl.BlockSpec((1, D, F), lambda i: (i, 0, 0)),
                  pl.BlockSpec((1, 1, F), lambda i: (i, 0, 0)),
                  pl.BlockSpec((1, F, D), lambda i: (i, 0, 0)),
                  pl.BlockSpec((1, 1, D), lambda i: (i, 0, 0))],
        out_specs=pl.BlockSpec((1, CAPP, D), lambda i: (i, 0, 0)),
        out_shape=jax.ShapeDtypeStruct((EE, CAPP, D), _f32),
    )(X, W1, b1.reshape(EE, 1, F), W2, b2.reshape(EE, 1, D))


def _epi_body(h_ref, r1_ref, r2_ref, g1_ref, g2_ref, o_ref):
    g1 = g1_ref[...].astype(_bf16).astype(_f32)[:, None]
    g2 = g2_ref[...].astype(_bf16).astype(_f32)[:, None]
    r1 = r1_ref[...].astype(_bf16).astype(_f32)
    r2 = r2_ref[...].astype(_bf16).astype(_f32)
    t1 = jnp.where(g1 > 0, g1 * r1, 0.0)
    t2 = jnp.where(g2 > 0, g2 * r2, 0.0)
    o_ref[...] = h_ref[...] + (t1 + t2)


def _epi(hf, r1, r2, g1, g2):
    row = pl.BlockSpec((256, D), lambda i: (i, 0))
    gv = pl.BlockSpec((256,), lambda i: (i,))
    return pl.pallas_call(
        _epi_body,
        grid=(NT // 256,),
        in_specs=[row, row, row, gv, gv],
        out_specs=row,
        out_shape=jax.ShapeDtypeStruct((NT, D), _f32),
    )(hf, r1, r2, g1, g2)


def _final_body(hn_ref, wc_ref, bc_ref, o_ref):
    pooled = jnp.sum(hn_ref[...], axis=1) / float(SV)
    o_ref[...] = _dot(pooled, wc_ref[...]) + bc_ref[...]


def _final(hn, Wc, bc):
    return pl.pallas_call(
        _final_body,
        out_shape=jax.ShapeDtypeStruct((BB, NCLSP), _f32),
    )(hn, Wc, bc)


@functools.lru_cache(maxsize=None)
def _build_dispatch():
    info = plsc.get_sparse_core_info()
    nc = info.num_cores
    mesh = plsc.VectorSubcoreMesh(core_axis_name="c", subcore_axis_name="s")

    @functools.partial(
        pl.kernel, mesh=mesh,
        out_type=jax.ShapeDtypeStruct((XR, D), _f32),
        scratch_types=[pltpu.VMEM((TPW, D), _f32),
                       pltpu.VMEM((TPW,), jnp.int32),
                       pltpu.VMEM((TPW,), jnp.int32),
                       pltpu.SemaphoreType.DMA],
    )
    def dispatch(hn2, d1, d2, xout, rows_v, i1_v, i2_v, sem):
        wid = lax.axis_index("s") * nc + lax.axis_index("c")
        base = wid * TPW
        pltpu.sync_copy(hn2.at[pl.ds(base, TPW)], rows_v)
        pltpu.sync_copy(d1.at[pl.ds(base, TPW)], i1_v)
        pltpu.sync_copy(d2.at[pl.ds(base, TPW)], i2_v)
        pltpu.async_copy(rows_v, xout.at[i1_v], sem).wait()
        pltpu.async_copy(rows_v, xout.at[i2_v], sem).wait()

    return dispatch


@functools.lru_cache(maxsize=None)
def _build_combine():
    info = plsc.get_sparse_core_info()
    nc = info.num_cores
    mesh = plsc.VectorSubcoreMesh(core_axis_name="c", subcore_axis_name="s")

    @functools.partial(
        pl.kernel, mesh=mesh,
        out_type=(jax.ShapeDtypeStruct((NT, D), _f32),
                  jax.ShapeDtypeStruct((NT, D), _f32)),
        scratch_types=[pltpu.VMEM((TPW,), jnp.int32),
                       pltpu.VMEM((TPW, D), _f32),
                       pltpu.SemaphoreType.DMA],
    )
    def combine(eo, c1, c2, r1, r2, i_v, rows_v, sem):
        wid = lax.axis_index("s") * nc + lax.axis_index("c")
        base = wid * TPW
        pltpu.sync_copy(c1.at[pl.ds(base, TPW)], i_v)
        pltpu.async_copy(eo.at[i_v], rows_v, sem).wait()
        pltpu.sync_copy(rows_v, r1.at[pl.ds(base, TPW)])
        pltpu.sync_copy(c2.at[pl.ds(base, TPW)], i_v)
        pltpu.async_copy(eo.at[i_v], rows_v, sem).wait()
        pltpu.sync_copy(rows_v, r2.at[pl.ds(base, TPW)])

    return combine


def _dispatch(hn2, d1, d2):
    return _build_dispatch()(hn2, d1, d2)


def _combine(eo, c1, c2):
    return _build_combine()(eo, c1, c2)


def _pad_tokens(x197):
    return jnp.pad(x197, ((0, 0), (0, SP - SV), (0, 0))).reshape(NT, D)


def kernel(x, Wpatch, bpatch, cls_tok, pos_emb, ln1_s, ln1_b, ln2_s, ln2_b,
           Wq, bq, Wk, bk, Wv, bv, Wo, bo, gate_w, W1, b1, W2, b2,
           lnf_s, lnf_b, Wc, bc):
    g = 224 // 16
    patches = x.reshape(BB, 3, g, 16, g, 16).transpose(
        0, 2, 4, 1, 3, 5).reshape(BB * g * g, D)
    emb = _embed(patches, Wpatch, bpatch).reshape(BB, g * g, D)
    h = jnp.concatenate(
        [jnp.broadcast_to(cls_tok, (BB, 1, D)), emb], 1) + pos_emb

    Wc_p = jnp.pad(Wc, ((0, 0), (0, NCLSP - NCLS)))
    bc_p = jnp.pad(bc, ((0, NCLSP - NCLS),))

    for i in range(4):
        hn1 = _ln(h, ln1_s[i], ln1_b[i])
        q, k, v = _qkv(_pad_tokens(hn1), Wq, bq, Wk, bk, Wv, bv)
        o = _attncore(q, k, v).reshape(NT, D)
        hf = _proj(_pad_tokens(h), o, Wo, bo)
        h = hf.reshape(BB, SP, D)[:, :SV]
        hn2 = _ln(h, ln2_s[i], ln2_b[i])
        hn2f = _pad_tokens(hn2)
        lg = _gate(hn2f, gate_w)
        d1, d2, c1, c2, g1, g2 = _rgb(lg)
        X = _dispatch(hn2f, d1, d2)
        eo = _ffn(X[:NSLOT].reshape(EE, CAPP, D), W1, b1, W2, b2)
        r1, r2 = _combine(eo.reshape(NSLOT, D), c1, c2)
        hf = _epi(_pad_tokens(h), r1, r2, g1, g2)
        h = hf.reshape(BB, SP, D)[:, :SV]

    hn = _ln(h, lnf_s, lnf_b)
    out = _final(hn, Wc_p, bc_p)
    return out[:, :NCLS]
